# baseline (speedup 1.0000x reference)
import dataclasses

import jax
import jax.numpy as jnp
from jax import lax
from jax.experimental import pallas as pl
from jax.experimental.pallas import tpu as pltpu
from jax.experimental.pallas import tpu_sc as plsc

NU = 25000
NI = 25000
N = NU + NI
H = 16
NLAYER = 3
E = 800000

NTILES = 16
CH = 128
GRP = 8
NGRP = 50
EP = NTILES * NGRP * GRP * CH
EROWS_T = NGRP * GRP

RCH = 26
NODES_T = RCH * CH
NP = NTILES * NODES_T
S = 2 * NP

_mesh = plsc.VectorSubcoreMesh(core_axis_name="c", subcore_axis_name="s")

_cparams = pltpu.CompilerParams()
if "needs_layout_passes" in pltpu.CompilerParams.__dataclass_fields__:
    _cparams = dataclasses.replace(_cparams, needs_layout_passes=False)
if "use_tc_tiling_on_sc" in pltpu.CompilerParams.__dataclass_fields__:
    _cparams = dataclasses.replace(_cparams, use_tc_tiling_on_sc=False)

_f32 = jnp.float32
_i32 = jnp.int32


def _fill(ref, val, n):
    v = jnp.full((16,), val)

    @pl.loop(0, n, step=16)
    def _(k):
        ref[pl.ds(k, 16)] = v


def _rsqrt16(d):
    bits = plsc.bitcast(d, _i32)
    y = plsc.bitcast(jnp.full((16,), 0x5F3759DF, _i32)
                     - lax.shift_right_logical(bits, 1), _f32)
    for _ in range(3):
        y = y * (1.5 - 0.5 * d * y * y)
    return jnp.where(d > 0.5, y, 0.0)


def _k23_body(xa_h, xb_h, attp, srcs, dst2,
              oa_h, ob_h,
              x0a, x1a, y0a, y1a, o0a, o1a,
              x0b, x1b, y0b, y1b, o0b, o1b,
              z_sp, deg_sp, sbuf, dbuf, stage, dist, wb, ones_b, degb,
              zin, xin, oin, xob, yob, oob,
              semi, semg, sems, semin, semout):
    c = lax.axis_index("c")
    s = lax.axis_index("s")
    r0 = (c * NTILES + s) * NODES_T
    nb = s * NODES_T
    er = s * EROWS_T

    _fill(degb, 0.0, CH)
    _fill(ones_b, 1.0, CH)

    @pl.loop(0, RCH)
    def _(i):
        pltpu.async_copy(degb, deg_sp.at[pl.ds(nb + i * CH, CH)],
                         semg.at[0])

    @pl.loop(0, RCH)
    def _(i):
        pltpu.make_async_copy(degb, deg_sp.at[pl.ds(nb, CH)],
                              semg.at[0]).wait()

    plsc.subcore_barrier()

    def d_issue(ss, g):
        pltpu.async_copy(dst2.at[pl.ds(er + g * GRP, GRP)], dbuf.at[ss],
                         semi.at[ss])

    def d_wait(ss):
        pltpu.make_async_copy(dst2.at[pl.ds(er, GRP)], dbuf.at[ss],
                              semi.at[ss]).wait()

    def sc_issue(ss):
        for j in range(GRP):
            pltpu.async_copy(ones_b, deg_sp.at[dbuf.at[ss, j]],
                             sems.at[ss], add=True)

    def sc_drain(ss):
        for j in range(GRP):
            pltpu.make_async_copy(ones_b, deg_sp.at[dbuf.at[ss, j]],
                                  sems.at[ss]).wait()

    d_issue(0, 0)

    @pl.loop(0, NGRP, step=2)
    def _(g):
        d_issue(1, g + 1)
        d_wait(0)
        sc_issue(0)
        sc_drain(0)

        @pl.when(g + 2 < NGRP)
        def _():
            d_issue(0, g + 2)

        d_wait(1)
        sc_issue(1)
        sc_drain(1)

    plsc.subcore_barrier()

    @pl.loop(0, RCH)
    def _(i):
        pltpu.sync_copy(deg_sp.at[pl.ds(nb + i * CH, CH)], degb)

        @pl.loop(0, CH, step=16)
        def _(k):
            dist[pl.ds(i * CH + k, 16)] = _rsqrt16(degb[pl.ds(k, 16)])

    pltpu.sync_copy(attp, wb)
    _v = wb[pl.ds(0, 16)]
    _e = jnp.exp(_v - jnp.max(_v))
    wall = _e / jnp.sum(_e)

    def ew_init(x_h, y_o, out_o):
        wv = jnp.full((16,), wall[0])

        def in_issue(p, i):
            pltpu.async_copy(x_h.at[pl.ds(r0 + i * CH, CH)], xin.at[p],
                             semin.at[p])

        def in_wait(p):
            pltpu.make_async_copy(x_h.at[pl.ds(r0, CH)], xin.at[p],
                                  semin.at[p]).wait()

        def compute(p, i):
            @pl.loop(0, CH, step=16)
            def _(rk):
                dvv = dist[pl.ds(i * CH + rk, 16)]
                for t in range(16):
                    r = rk + t
                    dv = jnp.full((16,), dvv[t])
                    x = xin[p, r, pl.ds(0, 16)]
                    yob[p, r, pl.ds(0, 16)] = dv * x
                    oob[p, r, pl.ds(0, 16)] = wv * x

        def out_issue(p, i):
            pltpu.async_copy(yob.at[p], y_o.at[pl.ds(r0 + i * CH, CH)],
                             semout.at[p])
            pltpu.async_copy(oob.at[p], out_o.at[pl.ds(r0 + i * CH, CH)],
                             semout.at[p])

        def out_wait(p):
            for _ in range(2):
                pltpu.make_async_copy(yob.at[p], y_o.at[pl.ds(r0, CH)],
                                      semout.at[p]).wait()

        in_issue(0, 0)

        @pl.loop(0, RCH, step=2)
        def _(i):
            in_issue(1, i + 1)
            in_wait(0)

            @pl.when(i >= 2)
            def _():
                out_wait(0)

            compute(0, i)
            out_issue(0, i)

            @pl.when(i + 2 < RCH)
            def _():
                in_issue(0, i + 2)

            in_wait(1)

            @pl.when(i >= 2)
            def _():
                out_wait(1)

            compute(1, i + 1)
            out_issue(1, i + 1)

        out_wait(0)
        out_wait(1)

    def layer_pass(lidx, y_h, x_h, out_h, xo_h, yo_h, oo_h):
        last = xo_h is None
        wv = jnp.full((16,), wall[lidx])

        @pl.loop(0, CH)
        def _(r):
            stage[0, r, pl.ds(0, 16)] = jnp.zeros((16,), _f32)

        @pl.loop(0, RCH)
        def _(i):
            pltpu.async_copy(stage.at[0, pl.ds(0, CH)],
                             z_sp.at[pl.ds(nb + i * CH, CH)], semg.at[0])

        @pl.loop(0, RCH)
        def _(i):
            pltpu.make_async_copy(stage.at[0, pl.ds(0, CH)],
                                  z_sp.at[pl.ds(nb, CH)], semg.at[0]).wait()

        plsc.subcore_barrier()

        def idx_issue(ss, g):
            row = er + g * GRP
            pltpu.async_copy(srcs.at[c, pl.ds(row, GRP)], sbuf.at[ss],
                             semi.at[ss])
            pltpu.async_copy(dst2.at[pl.ds(row, GRP)], dbuf.at[ss],
                             semi.at[ss])

        def idx_wait(ss):
            for _ in range(2):
                pltpu.make_async_copy(dst2.at[pl.ds(er, GRP)], dbuf.at[ss],
                                      semi.at[ss]).wait()

        def g_issue(ss):
            for j in range(GRP):
                pltpu.async_copy(y_h.at[sbuf.at[ss, j]],
                                 stage.at[ss, pl.ds(j * CH, CH)],
                                 semg.at[ss])

        def g_drain(ss):
            for j in range(GRP):
                pltpu.make_async_copy(y_h.at[sbuf.at[ss, j]],
                                      stage.at[ss, pl.ds(j * CH, CH)],
                                      semg.at[ss]).wait()

        def s_issue(ss):
            for j in range(GRP):
                pltpu.async_copy(stage.at[ss, pl.ds(j * CH, CH)],
                                 z_sp.at[dbuf.at[ss, j]], sems.at[ss],
                                 add=True)

        def s_drain(ss):
            for j in range(GRP):
                pltpu.make_async_copy(stage.at[ss, pl.ds(j * CH, CH)],
                                      z_sp.at[dbuf.at[ss, j]],
                                      sems.at[ss]).wait()

        idx_issue(0, 0)

        @pl.loop(0, NGRP, step=2)
        def _(g):
            idx_issue(1, g + 1)
            idx_wait(0)
            g_issue(0)
            g_drain(0)
            s_issue(0)
            idx_wait(1)
            g_issue(1)
            s_drain(0)

            @pl.when(g + 2 < NGRP)
            def _():
                idx_issue(0, g + 2)

            g_drain(1)
            s_issue(1)
            s_drain(1)

        plsc.subcore_barrier()

        def in_issue(p, i):
            off = i * CH
            pltpu.async_copy(x_h.at[pl.ds(r0 + off, CH)], xin.at[p],
                             semin.at[p])
            pltpu.async_copy(out_h.at[pl.ds(r0 + off, CH)], oin.at[p],
                             semin.at[p])

        def in_wait(p, i):
            pltpu.sync_copy(z_sp.at[pl.ds(nb + i * CH, CH)], zin.at[p])
            for _ in range(2):
                pltpu.make_async_copy(x_h.at[pl.ds(r0, CH)], xin.at[p],
                                      semin.at[p]).wait()

        def compute(p, i):
            @pl.loop(0, CH, step=16)
            def _(rk):
                dvv = dist[pl.ds(i * CH + rk, 16)]
                for t in range(16):
                    r = rk + t
                    dv = jnp.full((16,), dvv[t])
                    z = zin[p, r, pl.ds(0, 16)]
                    x = xin[p, r, pl.ds(0, 16)]
                    o = oin[p, r, pl.ds(0, 16)]
                    xn = x + dv * z
                    if not last:
                        xob[p, r, pl.ds(0, 16)] = xn
                        yob[p, r, pl.ds(0, 16)] = dv * xn
                    oob[p, r, pl.ds(0, 16)] = o + wv * xn

        def out_issue(p, i):
            off = r0 + i * CH
            if not last:
                pltpu.async_copy(xob.at[p], xo_h.at[pl.ds(off, CH)],
                                 semout.at[p])
                pltpu.async_copy(yob.at[p], yo_h.at[pl.ds(off, CH)],
                                 semout.at[p])
            pltpu.async_copy(oob.at[p], oo_h.at[pl.ds(off, CH)],
                             semout.at[p])

        def out_wait(p):
            for _ in range(1 if last else 3):
                pltpu.make_async_copy(oob.at[p], oo_h.at[pl.ds(r0, CH)],
                                      semout.at[p]).wait()

        in_issue(0, 0)

        @pl.loop(0, RCH, step=2)
        def _(i):
            in_issue(1, i + 1)
            in_wait(0, i)

            @pl.when(i >= 2)
            def _():
                out_wait(0)

            compute(0, i)
            out_issue(0, i)

            @pl.when(i + 2 < RCH)
            def _():
                in_issue(0, i + 2)

            in_wait(1, i + 1)

            @pl.when(i >= 2)
            def _():
                out_wait(1)

            compute(1, i + 1)
            out_issue(1, i + 1)

        out_wait(0)
        out_wait(1)

        plsc.subcore_barrier()

    ew_init(xa_h, y0a, o0a)
    ew_init(xb_h, y0b, o0b)
    plsc.subcore_barrier()
    layer_pass(1, y0a, xa_h, o0a, x1a, y1a, o1a)
    layer_pass(1, y0b, xb_h, o0b, x1b, y1b, o1b)
    layer_pass(2, y1a, x1a, o1a, x0a, y0a, o0a)
    layer_pass(2, y1b, x1b, o1b, x0b, y0b, o0b)
    layer_pass(3, y0a, x0a, o0a, None, None, oa_h)
    layer_pass(3, y0b, x0b, o0b, None, None, ob_h)


def _run_k23(xa, xb, attp, srcs, dst2):
    sds = jax.ShapeDtypeStruct((S, H), _f32)
    return pl.kernel(
        _k23_body,
        out_type=(sds,) * 14,
        mesh=_mesh,
        compiler_params=_cparams,
        scratch_types=[
            pltpu.VMEM_SHARED((NP, H), _f32),
            pltpu.VMEM_SHARED((NP,), _f32),
            pltpu.VMEM((2, GRP, CH), _i32),
            pltpu.VMEM((2, GRP, CH), _i32),
            pltpu.VMEM((2, GRP * CH, H), _f32),
            pltpu.VMEM((NODES_T,), _f32),
            pltpu.VMEM((16,), _f32),
            pltpu.VMEM((CH,), _f32),
            pltpu.VMEM((CH,), _f32),
            pltpu.VMEM((2, CH, H), _f32),
            pltpu.VMEM((2, CH, H), _f32),
            pltpu.VMEM((2, CH, H), _f32),
            pltpu.VMEM((2, CH, H), _f32),
            pltpu.VMEM((2, CH, H), _f32),
            pltpu.VMEM((2, CH, H), _f32),
            pltpu.SemaphoreType.DMA((2,)),
            pltpu.SemaphoreType.DMA((2,)),
            pltpu.SemaphoreType.DMA((2,)),
            pltpu.SemaphoreType.DMA((2,)),
            pltpu.SemaphoreType.DMA((2,)),
        ],
    )(xa, xb, attp, srcs, dst2)


def kernel(edge_index, user_emb, item_emb, attention):
    ei = edge_index.astype(_i32)
    pad = jnp.full((EP - E,), NP - 1, _i32)
    srca = jnp.concatenate([ei[0], pad]).reshape(-1, CH)
    srcs = jnp.stack([srca, srca + NP])
    dst2 = jnp.concatenate([ei[1], pad]).reshape(-1, CH)

    all_emb = jnp.concatenate([user_emb, item_emb], axis=0)
    padn = ((0, NP - N), (0, 0))
    xa = jnp.concatenate([jnp.pad(all_emb[:, 0:16], padn),
                          jnp.pad(all_emb[:, 16:32], padn)], axis=0)
    xb = jnp.concatenate([jnp.pad(all_emb[:, 32:48], padn),
                          jnp.pad(all_emb[:, 48:64], padn)], axis=0)

    attp = jnp.pad(attention.reshape(-1).astype(_f32), (0, 12),
                   constant_values=-1e30)

    outs = _run_k23(xa, xb, attp, srcs, dst2)
    outa, outb = outs[0], outs[1]

    final = jnp.concatenate([outa[:N], outa[NP:NP + N],
                             outb[:N], outb[NP:NP + N]], axis=1)
    return (final[:NU], final[NU:])


# speedup vs baseline: 1.0940x; 1.0940x over previous
import dataclasses

import jax
import jax.numpy as jnp
from jax import lax
from jax.experimental import pallas as pl
from jax.experimental.pallas import tpu as pltpu
from jax.experimental.pallas import tpu_sc as plsc

NU = 25000
NI = 25000
N = NU + NI
H = 16
NLAYER = 3
E = 800000

NTILES = 16
CH = 128
GRP = 8
NGRP = 50
EP = NTILES * NGRP * GRP * CH
EROWS_T = NGRP * GRP

RCH = 26
NODES_T = RCH * CH
NP = NTILES * NODES_T
S = 2 * NP

_mesh = plsc.VectorSubcoreMesh(core_axis_name="c", subcore_axis_name="s")

_cparams = pltpu.CompilerParams()
if "needs_layout_passes" in pltpu.CompilerParams.__dataclass_fields__:
    _cparams = dataclasses.replace(_cparams, needs_layout_passes=False)
if "use_tc_tiling_on_sc" in pltpu.CompilerParams.__dataclass_fields__:
    _cparams = dataclasses.replace(_cparams, use_tc_tiling_on_sc=False)

_f32 = jnp.float32
_i32 = jnp.int32


def _fill(ref, val, n):
    v = jnp.full((16,), val)

    @pl.loop(0, n, step=16)
    def _(k):
        ref[pl.ds(k, 16)] = v


def _rsqrt16(d):
    bits = plsc.bitcast(d, _i32)
    y = plsc.bitcast(jnp.full((16,), 0x5F3759DF, _i32)
                     - lax.shift_right_logical(bits, 1), _f32)
    for _ in range(3):
        y = y * (1.5 - 0.5 * d * y * y)
    return jnp.where(d > 0.5, y, 0.0)


def _k1_body(dst2, attp, dis_o, w_o, deg_sp, dbuf, ones_b, degb, disb, abuf,
             semd, sems):
    c = lax.axis_index("c")
    s = lax.axis_index("s")
    nb = s * NODES_T
    er = s * EROWS_T

    @pl.when(c == 0)
    def _():
        _fill(disb, 0.0, CH)
        _fill(ones_b, 1.0, CH)

        @pl.loop(0, RCH)
        def _(i):
            pltpu.sync_copy(disb, deg_sp.at[pl.ds(nb + i * CH, CH)])

    plsc.subcore_barrier()

    @pl.when(c == 0)
    def _():
        def d_issue(ss, g):
            pltpu.async_copy(dst2.at[pl.ds(er + g * GRP, GRP)], dbuf.at[ss],
                             semd.at[ss])

        def d_wait(ss):
            pltpu.make_async_copy(dst2.at[pl.ds(er, GRP)], dbuf.at[ss],
                                  semd.at[ss]).wait()

        def sc_issue(ss):
            for j in range(GRP):
                pltpu.async_copy(ones_b, deg_sp.at[dbuf.at[ss, j]],
                                 sems.at[ss], add=True)

        def sc_drain(ss):
            for j in range(GRP):
                pltpu.make_async_copy(ones_b, deg_sp.at[dbuf.at[ss, j]],
                                      sems.at[ss]).wait()

        d_issue(0, 0)

        @pl.loop(0, NGRP, step=2)
        def _(g):
            d_issue(1, g + 1)
            d_wait(0)
            sc_issue(0)
            sc_drain(0)

            @pl.when(g + 2 < NGRP)
            def _():
                d_issue(0, g + 2)

            d_wait(1)
            sc_issue(1)
            sc_drain(1)

    plsc.subcore_barrier()

    @pl.when(c == 0)
    def _():
        @pl.loop(0, RCH)
        def _(i):
            off = nb + i * CH
            pltpu.sync_copy(deg_sp.at[pl.ds(off, CH)], degb)

            @pl.loop(0, CH, step=16)
            def _(k):
                disb[pl.ds(k, 16)] = _rsqrt16(degb[pl.ds(k, 16)])

            pltpu.sync_copy(disb, dis_o.at[pl.ds(off, CH)])

        @pl.when(s == 0)
        def _():
            pltpu.sync_copy(attp, abuf)
            v = abuf[...]
            e = jnp.exp(v - jnp.max(v))
            abuf[...] = e / jnp.sum(e)
            pltpu.sync_copy(abuf, w_o)


def _run_k1(dst2, attp):
    return pl.kernel(
        _k1_body,
        out_type=(jax.ShapeDtypeStruct((NP,), _f32),
                  jax.ShapeDtypeStruct((16,), _f32)),
        mesh=_mesh,
        compiler_params=_cparams,
        scratch_types=[
            pltpu.VMEM_SHARED((NP,), _f32),
            pltpu.VMEM((2, GRP, CH), _i32),
            pltpu.VMEM((CH,), _f32),
            pltpu.VMEM((CH,), _f32),
            pltpu.VMEM((CH,), _f32),
            pltpu.VMEM((16,), _f32),
            pltpu.SemaphoreType.DMA((2,)),
            pltpu.SemaphoreType.DMA((2,)),
        ],
    )(dst2, attp)


def _k23_body(xa_h, xb_h, dis_h, w_h, srcs, dst2,
              oa_h, ob_h,
              x0a, x1a, y0a, y1a, o0a, o1a,
              x0b, x1b, y0b, y1b, o0b, o1b,
              z_sp, sbuf, dbuf, stage, dist, wb,
              zin, xin, oin, xob, yob, oob,
              semi, semg, sems, semin, semout):
    c = lax.axis_index("c")
    s = lax.axis_index("s")
    r0 = (c * NTILES + s) * NODES_T
    nb = s * NODES_T
    er = s * EROWS_T

    pltpu.sync_copy(dis_h.at[pl.ds(nb, NODES_T)], dist)
    pltpu.sync_copy(w_h, wb)
    wall = wb[pl.ds(0, 16)]

    def ew_init(x_h, y_o, out_o):
        wv = jnp.full((16,), wall[0])

        def in_issue(p, i):
            pltpu.async_copy(x_h.at[pl.ds(r0 + i * CH, CH)], xin.at[p],
                             semin.at[p])

        def in_wait(p):
            pltpu.make_async_copy(x_h.at[pl.ds(r0, CH)], xin.at[p],
                                  semin.at[p]).wait()

        def compute(p, i):
            @pl.loop(0, CH, step=16)
            def _(rk):
                dvv = dist[pl.ds(i * CH + rk, 16)]
                for t in range(16):
                    r = rk + t
                    dv = jnp.full((16,), dvv[t])
                    x = xin[p, r, pl.ds(0, 16)]
                    yob[p, r, pl.ds(0, 16)] = dv * x
                    oob[p, r, pl.ds(0, 16)] = wv * x

        def out_issue(p, i):
            pltpu.async_copy(yob.at[p], y_o.at[pl.ds(r0 + i * CH, CH)],
                             semout.at[p])
            pltpu.async_copy(oob.at[p], out_o.at[pl.ds(r0 + i * CH, CH)],
                             semout.at[p])

        def out_wait(p):
            for _ in range(2):
                pltpu.make_async_copy(yob.at[p], y_o.at[pl.ds(r0, CH)],
                                      semout.at[p]).wait()

        in_issue(0, 0)

        @pl.loop(0, RCH, step=2)
        def _(i):
            in_issue(1, i + 1)
            in_wait(0)

            @pl.when(i >= 2)
            def _():
                out_wait(0)

            compute(0, i)
            out_issue(0, i)

            @pl.when(i + 2 < RCH)
            def _():
                in_issue(0, i + 2)

            in_wait(1)

            @pl.when(i >= 2)
            def _():
                out_wait(1)

            compute(1, i + 1)
            out_issue(1, i + 1)

        out_wait(0)
        out_wait(1)

    def layer_pass(lidx, y_h, x_h, out_h, xo_h, yo_h, oo_h):
        last = xo_h is None
        wv = jnp.full((16,), wall[lidx])

        @pl.loop(0, CH)
        def _(r):
            stage[0, r, pl.ds(0, 16)] = jnp.zeros((16,), _f32)

        @pl.loop(0, RCH)
        def _(i):
            pltpu.async_copy(stage.at[0, pl.ds(0, CH)],
                             z_sp.at[pl.ds(nb + i * CH, CH)], semg.at[0])

        @pl.loop(0, RCH)
        def _(i):
            pltpu.make_async_copy(stage.at[0, pl.ds(0, CH)],
                                  z_sp.at[pl.ds(nb, CH)], semg.at[0]).wait()

        plsc.subcore_barrier()

        def idx_issue(ss, g):
            row = er + g * GRP
            pltpu.async_copy(srcs.at[c, pl.ds(row, GRP)], sbuf.at[ss],
                             semi.at[ss])
            pltpu.async_copy(dst2.at[pl.ds(row, GRP)], dbuf.at[ss],
                             semi.at[ss])

        def idx_wait(ss):
            for _ in range(2):
                pltpu.make_async_copy(dst2.at[pl.ds(er, GRP)], dbuf.at[ss],
                                      semi.at[ss]).wait()

        def g_issue(ss):
            for j in range(GRP):
                pltpu.async_copy(y_h.at[sbuf.at[ss, j]],
                                 stage.at[ss, pl.ds(j * CH, CH)],
                                 semg.at[ss])

        def g_drain(ss):
            for j in range(GRP):
                pltpu.make_async_copy(y_h.at[sbuf.at[ss, j]],
                                      stage.at[ss, pl.ds(j * CH, CH)],
                                      semg.at[ss]).wait()

        def s_issue(ss):
            for j in range(GRP):
                pltpu.async_copy(stage.at[ss, pl.ds(j * CH, CH)],
                                 z_sp.at[dbuf.at[ss, j]], sems.at[ss],
                                 add=True)

        def s_drain(ss):
            for j in range(GRP):
                pltpu.make_async_copy(stage.at[ss, pl.ds(j * CH, CH)],
                                      z_sp.at[dbuf.at[ss, j]],
                                      sems.at[ss]).wait()

        idx_issue(0, 0)

        @pl.loop(0, NGRP, step=2)
        def _(g):
            idx_issue(1, g + 1)
            idx_wait(0)
            g_issue(0)
            g_drain(0)
            s_issue(0)
            idx_wait(1)
            g_issue(1)
            s_drain(0)

            @pl.when(g + 2 < NGRP)
            def _():
                idx_issue(0, g + 2)

            g_drain(1)
            s_issue(1)
            s_drain(1)

        plsc.subcore_barrier()

        def in_issue(p, i):
            off = i * CH
            pltpu.async_copy(x_h.at[pl.ds(r0 + off, CH)], xin.at[p],
                             semin.at[p])
            pltpu.async_copy(out_h.at[pl.ds(r0 + off, CH)], oin.at[p],
                             semin.at[p])

        def in_wait(p, i):
            pltpu.sync_copy(z_sp.at[pl.ds(nb + i * CH, CH)], zin.at[p])
            for _ in range(2):
                pltpu.make_async_copy(x_h.at[pl.ds(r0, CH)], xin.at[p],
                                      semin.at[p]).wait()

        def compute(p, i):
            @pl.loop(0, CH, step=16)
            def _(rk):
                dvv = dist[pl.ds(i * CH + rk, 16)]
                for t in range(16):
                    r = rk + t
                    dv = jnp.full((16,), dvv[t])
                    z = zin[p, r, pl.ds(0, 16)]
                    x = xin[p, r, pl.ds(0, 16)]
                    o = oin[p, r, pl.ds(0, 16)]
                    xn = x + dv * z
                    if not last:
                        xob[p, r, pl.ds(0, 16)] = xn
                        yob[p, r, pl.ds(0, 16)] = dv * xn
                    oob[p, r, pl.ds(0, 16)] = o + wv * xn

        def out_issue(p, i):
            off = r0 + i * CH
            if not last:
                pltpu.async_copy(xob.at[p], xo_h.at[pl.ds(off, CH)],
                                 semout.at[p])
                pltpu.async_copy(yob.at[p], yo_h.at[pl.ds(off, CH)],
                                 semout.at[p])
            pltpu.async_copy(oob.at[p], oo_h.at[pl.ds(off, CH)],
                             semout.at[p])

        def out_wait(p):
            for _ in range(1 if last else 3):
                pltpu.make_async_copy(oob.at[p], oo_h.at[pl.ds(r0, CH)],
                                      semout.at[p]).wait()

        in_issue(0, 0)

        @pl.loop(0, RCH, step=2)
        def _(i):
            in_issue(1, i + 1)
            in_wait(0, i)

            @pl.when(i >= 2)
            def _():
                out_wait(0)

            compute(0, i)
            out_issue(0, i)

            @pl.when(i + 2 < RCH)
            def _():
                in_issue(0, i + 2)

            in_wait(1, i + 1)

            @pl.when(i >= 2)
            def _():
                out_wait(1)

            compute(1, i + 1)
            out_issue(1, i + 1)

        out_wait(0)
        out_wait(1)

        plsc.subcore_barrier()

    ew_init(xa_h, y0a, o0a)
    ew_init(xb_h, y0b, o0b)
    plsc.subcore_barrier()
    layer_pass(1, y0a, xa_h, o0a, x1a, y1a, o1a)
    layer_pass(1, y0b, xb_h, o0b, x1b, y1b, o1b)
    layer_pass(2, y1a, x1a, o1a, x0a, y0a, o0a)
    layer_pass(2, y1b, x1b, o1b, x0b, y0b, o0b)
    layer_pass(3, y0a, x0a, o0a, None, None, oa_h)
    layer_pass(3, y0b, x0b, o0b, None, None, ob_h)


def _run_k23(xa, xb, dis, w, srcs, dst2):
    sds = jax.ShapeDtypeStruct((S, H), _f32)
    return pl.kernel(
        _k23_body,
        out_type=(sds,) * 14,
        mesh=_mesh,
        compiler_params=_cparams,
        scratch_types=[
            pltpu.VMEM_SHARED((NP, H), _f32),
            pltpu.VMEM((2, GRP, CH), _i32),
            pltpu.VMEM((2, GRP, CH), _i32),
            pltpu.VMEM((2, GRP * CH, H), _f32),
            pltpu.VMEM((NODES_T,), _f32),
            pltpu.VMEM((16,), _f32),
            pltpu.VMEM((2, CH, H), _f32),
            pltpu.VMEM((2, CH, H), _f32),
            pltpu.VMEM((2, CH, H), _f32),
            pltpu.VMEM((2, CH, H), _f32),
            pltpu.VMEM((2, CH, H), _f32),
            pltpu.VMEM((2, CH, H), _f32),
            pltpu.SemaphoreType.DMA((2,)),
            pltpu.SemaphoreType.DMA((2,)),
            pltpu.SemaphoreType.DMA((2,)),
            pltpu.SemaphoreType.DMA((2,)),
            pltpu.SemaphoreType.DMA((2,)),
        ],
    )(xa, xb, dis, w, srcs, dst2)


def kernel(edge_index, user_emb, item_emb, attention):
    ei = edge_index.astype(_i32)
    pad = jnp.full((EP - E,), NP - 1, _i32)
    srca = jnp.concatenate([ei[0], pad]).reshape(-1, CH)
    srcs = jnp.stack([srca, srca + NP])
    dst2 = jnp.concatenate([ei[1], pad]).reshape(-1, CH)

    all_emb = jnp.concatenate([user_emb, item_emb], axis=0)
    padn = ((0, NP - N), (0, 0))
    xa = jnp.concatenate([jnp.pad(all_emb[:, 0:16], padn),
                          jnp.pad(all_emb[:, 16:32], padn)], axis=0)
    xb = jnp.concatenate([jnp.pad(all_emb[:, 32:48], padn),
                          jnp.pad(all_emb[:, 48:64], padn)], axis=0)

    attp = jnp.pad(attention.reshape(-1).astype(_f32), (0, 12),
                   constant_values=-1e30)

    dis, w = _run_k1(dst2, attp)
    outs = _run_k23(xa, xb, dis, w, srcs, dst2)
    outa, outb = outs[0], outs[1]

    final = jnp.concatenate([outa[:N], outa[NP:NP + N],
                             outb[:N], outb[NP:NP + N]], axis=1)
    return (final[:NU], final[NU:])


# speedup vs baseline: 1.1142x; 1.0184x over previous
import dataclasses

import jax
import jax.numpy as jnp
from jax import lax
from jax.experimental import pallas as pl
from jax.experimental.pallas import tpu as pltpu
from jax.experimental.pallas import tpu_sc as plsc

NU = 25000
NI = 25000
N = NU + NI
H = 16
NLAYER = 3
E = 800000

NTILES = 16
CH = 128
GRP = 10
NGRP = 40
EP = NTILES * NGRP * GRP * CH
EROWS_T = NGRP * GRP

RCH = 26
NODES_T = RCH * CH
NP = NTILES * NODES_T
S = 2 * NP

_mesh = plsc.VectorSubcoreMesh(core_axis_name="c", subcore_axis_name="s")

_cparams = pltpu.CompilerParams()
if "needs_layout_passes" in pltpu.CompilerParams.__dataclass_fields__:
    _cparams = dataclasses.replace(_cparams, needs_layout_passes=False)
if "use_tc_tiling_on_sc" in pltpu.CompilerParams.__dataclass_fields__:
    _cparams = dataclasses.replace(_cparams, use_tc_tiling_on_sc=False)

_f32 = jnp.float32
_i32 = jnp.int32


def _fill(ref, val, n):
    v = jnp.full((16,), val)

    @pl.loop(0, n, step=16)
    def _(k):
        ref[pl.ds(k, 16)] = v


def _rsqrt16(d):
    bits = plsc.bitcast(d, _i32)
    y = plsc.bitcast(jnp.full((16,), 0x5F3759DF, _i32)
                     - lax.shift_right_logical(bits, 1), _f32)
    for _ in range(3):
        y = y * (1.5 - 0.5 * d * y * y)
    return jnp.where(d > 0.5, y, 0.0)


def _k1_body(dst2, attp, dis_o, w_o, deg_sp, dbuf, ones_b, degb, disb, abuf,
             semd, sems):
    c = lax.axis_index("c")
    s = lax.axis_index("s")
    nb = s * NODES_T
    er = s * EROWS_T

    @pl.when(c == 0)
    def _():
        _fill(disb, 0.0, CH)
        _fill(ones_b, 1.0, CH)

        @pl.loop(0, RCH)
        def _(i):
            pltpu.sync_copy(disb, deg_sp.at[pl.ds(nb + i * CH, CH)])

    plsc.subcore_barrier()

    @pl.when(c == 0)
    def _():
        def d_issue(ss, g):
            pltpu.async_copy(dst2.at[pl.ds(er + g * GRP, GRP)], dbuf.at[ss],
                             semd.at[ss])

        def d_wait(ss):
            pltpu.make_async_copy(dst2.at[pl.ds(er, GRP)], dbuf.at[ss],
                                  semd.at[ss]).wait()

        def sc_issue(ss):
            for j in range(GRP):
                pltpu.async_copy(ones_b, deg_sp.at[dbuf.at[ss, j]],
                                 sems.at[ss], add=True)

        def sc_drain(ss):
            for j in range(GRP):
                pltpu.make_async_copy(ones_b, deg_sp.at[dbuf.at[ss, j]],
                                      sems.at[ss]).wait()

        d_issue(0, 0)

        @pl.loop(0, NGRP, step=2)
        def _(g):
            d_issue(1, g + 1)
            d_wait(0)
            sc_issue(0)
            sc_drain(0)

            @pl.when(g + 2 < NGRP)
            def _():
                d_issue(0, g + 2)

            d_wait(1)
            sc_issue(1)
            sc_drain(1)

    plsc.subcore_barrier()

    @pl.when(c == 0)
    def _():
        @pl.loop(0, RCH)
        def _(i):
            off = nb + i * CH
            pltpu.sync_copy(deg_sp.at[pl.ds(off, CH)], degb)

            @pl.loop(0, CH, step=16)
            def _(k):
                disb[pl.ds(k, 16)] = _rsqrt16(degb[pl.ds(k, 16)])

            pltpu.sync_copy(disb, dis_o.at[pl.ds(off, CH)])

        @pl.when(s == 0)
        def _():
            pltpu.sync_copy(attp, abuf)
            v = abuf[...]
            e = jnp.exp(v - jnp.max(v))
            abuf[...] = e / jnp.sum(e)
            pltpu.sync_copy(abuf, w_o)


def _run_k1(dst2, attp):
    return pl.kernel(
        _k1_body,
        out_type=(jax.ShapeDtypeStruct((NP,), _f32),
                  jax.ShapeDtypeStruct((16,), _f32)),
        mesh=_mesh,
        compiler_params=_cparams,
        scratch_types=[
            pltpu.VMEM_SHARED((NP,), _f32),
            pltpu.VMEM((2, GRP, CH), _i32),
            pltpu.VMEM((CH,), _f32),
            pltpu.VMEM((CH,), _f32),
            pltpu.VMEM((CH,), _f32),
            pltpu.VMEM((16,), _f32),
            pltpu.SemaphoreType.DMA((2,)),
            pltpu.SemaphoreType.DMA((2,)),
        ],
    )(dst2, attp)


def _k23_body(xa_h, xb_h, dis_h, w_h, srcs, dst2,
              oa_h, ob_h,
              x0a, x1a, y0a, y1a, o0a, o1a,
              x0b, x1b, y0b, y1b, o0b, o1b,
              z_sp, sbuf, dbuf, stage, dist, wb,
              zin, xin, oin, xob, yob, oob,
              semi, semg, sems, semin, semout):
    c = lax.axis_index("c")
    s = lax.axis_index("s")
    r0 = (c * NTILES + s) * NODES_T
    nb = s * NODES_T
    er = s * EROWS_T

    pltpu.sync_copy(dis_h.at[pl.ds(nb, NODES_T)], dist)
    pltpu.sync_copy(w_h, wb)
    wall = wb[pl.ds(0, 16)]

    def ew_init(x_h, y_o, out_o):
        wv = jnp.full((16,), wall[0])

        def in_issue(p, i):
            pltpu.async_copy(x_h.at[pl.ds(r0 + i * CH, CH)], xin.at[p],
                             semin.at[p])

        def in_wait(p):
            pltpu.make_async_copy(x_h.at[pl.ds(r0, CH)], xin.at[p],
                                  semin.at[p]).wait()

        def compute(p, i):
            @pl.loop(0, CH, step=16)
            def _(rk):
                dvv = dist[pl.ds(i * CH + rk, 16)]
                for t in range(16):
                    r = rk + t
                    dv = jnp.full((16,), dvv[t])
                    x = xin[p, r, pl.ds(0, 16)]
                    yob[p, r, pl.ds(0, 16)] = dv * x
                    oob[p, r, pl.ds(0, 16)] = wv * x

        def out_issue(p, i):
            pltpu.async_copy(yob.at[p], y_o.at[pl.ds(r0 + i * CH, CH)],
                             semout.at[p])
            pltpu.async_copy(oob.at[p], out_o.at[pl.ds(r0 + i * CH, CH)],
                             semout.at[p])

        def out_wait(p):
            for _ in range(2):
                pltpu.make_async_copy(yob.at[p], y_o.at[pl.ds(r0, CH)],
                                      semout.at[p]).wait()

        in_issue(0, 0)

        @pl.loop(0, RCH, step=2)
        def _(i):
            in_issue(1, i + 1)
            in_wait(0)

            @pl.when(i >= 2)
            def _():
                out_wait(0)

            compute(0, i)
            out_issue(0, i)

            @pl.when(i + 2 < RCH)
            def _():
                in_issue(0, i + 2)

            in_wait(1)

            @pl.when(i >= 2)
            def _():
                out_wait(1)

            compute(1, i + 1)
            out_issue(1, i + 1)

        out_wait(0)
        out_wait(1)

    def layer_pass(lidx, y_h, x_h, out_h, xo_h, yo_h, oo_h):
        last = xo_h is None
        wv = jnp.full((16,), wall[lidx])

        @pl.loop(0, CH)
        def _(r):
            stage[0, r, pl.ds(0, 16)] = jnp.zeros((16,), _f32)

        @pl.loop(0, RCH)
        def _(i):
            pltpu.async_copy(stage.at[0, pl.ds(0, CH)],
                             z_sp.at[pl.ds(nb + i * CH, CH)], semg.at[0])

        @pl.loop(0, RCH)
        def _(i):
            pltpu.make_async_copy(stage.at[0, pl.ds(0, CH)],
                                  z_sp.at[pl.ds(nb, CH)], semg.at[0]).wait()

        plsc.subcore_barrier()

        def idx_issue(ss, g):
            row = er + g * GRP
            pltpu.async_copy(srcs.at[c, pl.ds(row, GRP)], sbuf.at[ss],
                             semi.at[ss])
            pltpu.async_copy(dst2.at[pl.ds(row, GRP)], dbuf.at[ss],
                             semi.at[ss])

        def idx_wait(ss):
            for _ in range(2):
                pltpu.make_async_copy(dst2.at[pl.ds(er, GRP)], dbuf.at[ss],
                                      semi.at[ss]).wait()

        def g_issue(ss):
            for j in range(GRP):
                pltpu.async_copy(y_h.at[sbuf.at[ss, j]],
                                 stage.at[ss, pl.ds(j * CH, CH)],
                                 semg.at[ss])

        def g_drain(ss):
            for j in range(GRP):
                pltpu.make_async_copy(y_h.at[sbuf.at[ss, j]],
                                      stage.at[ss, pl.ds(j * CH, CH)],
                                      semg.at[ss]).wait()

        def s_issue(ss):
            for j in range(GRP):
                pltpu.async_copy(stage.at[ss, pl.ds(j * CH, CH)],
                                 z_sp.at[dbuf.at[ss, j]], sems.at[ss],
                                 add=True)

        def s_drain(ss):
            for j in range(GRP):
                pltpu.make_async_copy(stage.at[ss, pl.ds(j * CH, CH)],
                                      z_sp.at[dbuf.at[ss, j]],
                                      sems.at[ss]).wait()

        idx_issue(0, 0)

        @pl.loop(0, NGRP, step=2)
        def _(g):
            idx_issue(1, g + 1)
            idx_wait(0)
            g_issue(0)
            g_drain(0)
            s_issue(0)
            idx_wait(1)
            g_issue(1)
            s_drain(0)

            @pl.when(g + 2 < NGRP)
            def _():
                idx_issue(0, g + 2)

            g_drain(1)
            s_issue(1)
            s_drain(1)

        plsc.subcore_barrier()

        def in_issue(p, i):
            off = i * CH
            pltpu.async_copy(x_h.at[pl.ds(r0 + off, CH)], xin.at[p],
                             semin.at[p])
            pltpu.async_copy(out_h.at[pl.ds(r0 + off, CH)], oin.at[p],
                             semin.at[p])

        def in_wait(p, i):
            pltpu.sync_copy(z_sp.at[pl.ds(nb + i * CH, CH)], zin.at[p])
            for _ in range(2):
                pltpu.make_async_copy(x_h.at[pl.ds(r0, CH)], xin.at[p],
                                      semin.at[p]).wait()

        def compute(p, i):
            @pl.loop(0, CH, step=16)
            def _(rk):
                dvv = dist[pl.ds(i * CH + rk, 16)]
                for t in range(16):
                    r = rk + t
                    dv = jnp.full((16,), dvv[t])
                    z = zin[p, r, pl.ds(0, 16)]
                    x = xin[p, r, pl.ds(0, 16)]
                    o = oin[p, r, pl.ds(0, 16)]
                    xn = x + dv * z
                    if not last:
                        xob[p, r, pl.ds(0, 16)] = xn
                        yob[p, r, pl.ds(0, 16)] = dv * xn
                    oob[p, r, pl.ds(0, 16)] = o + wv * xn

        def out_issue(p, i):
            off = r0 + i * CH
            if not last:
                pltpu.async_copy(xob.at[p], xo_h.at[pl.ds(off, CH)],
                                 semout.at[p])
                pltpu.async_copy(yob.at[p], yo_h.at[pl.ds(off, CH)],
                                 semout.at[p])
            pltpu.async_copy(oob.at[p], oo_h.at[pl.ds(off, CH)],
                             semout.at[p])

        def out_wait(p):
            for _ in range(1 if last else 3):
                pltpu.make_async_copy(oob.at[p], oo_h.at[pl.ds(r0, CH)],
                                      semout.at[p]).wait()

        in_issue(0, 0)

        @pl.loop(0, RCH, step=2)
        def _(i):
            in_issue(1, i + 1)
            in_wait(0, i)

            @pl.when(i >= 2)
            def _():
                out_wait(0)

            compute(0, i)
            out_issue(0, i)

            @pl.when(i + 2 < RCH)
            def _():
                in_issue(0, i + 2)

            in_wait(1, i + 1)

            @pl.when(i >= 2)
            def _():
                out_wait(1)

            compute(1, i + 1)
            out_issue(1, i + 1)

        out_wait(0)
        out_wait(1)

        plsc.subcore_barrier()

    ew_init(xa_h, y0a, o0a)
    ew_init(xb_h, y0b, o0b)
    plsc.subcore_barrier()
    layer_pass(1, y0a, xa_h, o0a, x1a, y1a, o1a)
    layer_pass(1, y0b, xb_h, o0b, x1b, y1b, o1b)
    layer_pass(2, y1a, x1a, o1a, x0a, y0a, o0a)
    layer_pass(2, y1b, x1b, o1b, x0b, y0b, o0b)
    layer_pass(3, y0a, x0a, o0a, None, None, oa_h)
    layer_pass(3, y0b, x0b, o0b, None, None, ob_h)


def _run_k23(xa, xb, dis, w, srcs, dst2):
    sds = jax.ShapeDtypeStruct((S, H), _f32)
    return pl.kernel(
        _k23_body,
        out_type=(sds,) * 14,
        mesh=_mesh,
        compiler_params=_cparams,
        scratch_types=[
            pltpu.VMEM_SHARED((NP, H), _f32),
            pltpu.VMEM((2, GRP, CH), _i32),
            pltpu.VMEM((2, GRP, CH), _i32),
            pltpu.VMEM((2, GRP * CH, H), _f32),
            pltpu.VMEM((NODES_T,), _f32),
            pltpu.VMEM((16,), _f32),
            pltpu.VMEM((2, CH, H), _f32),
            pltpu.VMEM((2, CH, H), _f32),
            pltpu.VMEM((2, CH, H), _f32),
            pltpu.VMEM((2, CH, H), _f32),
            pltpu.VMEM((2, CH, H), _f32),
            pltpu.VMEM((2, CH, H), _f32),
            pltpu.SemaphoreType.DMA((2,)),
            pltpu.SemaphoreType.DMA((2,)),
            pltpu.SemaphoreType.DMA((2,)),
            pltpu.SemaphoreType.DMA((2,)),
            pltpu.SemaphoreType.DMA((2,)),
        ],
    )(xa, xb, dis, w, srcs, dst2)


def kernel(edge_index, user_emb, item_emb, attention):
    ei = edge_index.astype(_i32)
    pad = jnp.full((EP - E,), NP - 1, _i32)
    srca = jnp.concatenate([ei[0], pad]).reshape(-1, CH)
    srcs = jnp.stack([srca, srca + NP])
    dst2 = jnp.concatenate([ei[1], pad]).reshape(-1, CH)

    all_emb = jnp.concatenate([user_emb, item_emb], axis=0)
    padn = ((0, NP - N), (0, 0))
    xa = jnp.concatenate([jnp.pad(all_emb[:, 0:16], padn),
                          jnp.pad(all_emb[:, 16:32], padn)], axis=0)
    xb = jnp.concatenate([jnp.pad(all_emb[:, 32:48], padn),
                          jnp.pad(all_emb[:, 48:64], padn)], axis=0)

    attp = jnp.pad(attention.reshape(-1).astype(_f32), (0, 12),
                   constant_values=-1e30)

    dis, w = _run_k1(dst2, attp)
    outs = _run_k23(xa, xb, dis, w, srcs, dst2)
    outa, outb = outs[0], outs[1]

    final = jnp.concatenate([outa[:N], outa[NP:NP + N],
                             outb[:N], outb[NP:NP + N]], axis=1)
    return (final[:NU], final[NU:])
